# baseline (speedup 1.0000x reference)
import functools

import numpy as np

import jax
import jax.numpy as jnp
from jax import lax
from jax.experimental import pallas as pl
from jax.experimental.pallas import tpu as pltpu
from jax.experimental.pallas import tpu_sc as plsc

_NC = 2
_NS = 16
_NW = _NC * _NS

_B = 4096
_L = 200
_D = 64
_BLK = _B // _NW
_NBUF = 4


def _sc_body(xt_hbm, tok2_hbm, pos_hbm, out_hbm,
             idx_v, pos_v, gbuf0, gbuf1, gbuf2, gbuf3, tbuf,
             gsem0, gsem1, gsem2, gsem3, osem0, osem1):
    wid = lax.axis_index("s") * _NC + lax.axis_index("c")
    b0 = wid * _BLK
    tok_hbm = tok2_hbm

    pltpu.sync_copy(xt_hbm.at[:, pl.ds(b0, _BLK)], idx_v)
    pltpu.sync_copy(pos_hbm, pos_v)

    gbufs = (gbuf0, gbuf1, gbuf2, gbuf3)
    gsems = (gsem0, gsem1, gsem2, gsem3)
    osems = (osem0, osem1)

    lanes = lax.iota(jnp.int32, 16)
    dvecs = [16 * k + lanes for k in range(4)]

    def fire_gather(l, slot):
        @pl.when(l < _L)
        def _():
            pltpu.async_copy(tok_hbm.at[idx_v.at[l]], gbufs[slot],
                             gsems[slot])

    def substep(l, u):
        fire_gather(l + _NBUF - 1, (u + _NBUF - 1) % _NBUF)
        pv = [pos_v[l, pl.ds(16 * k, 16)] for k in range(4)]
        @pl.when(l >= 2)
        def _reclaim():
            for s in range(8):
                pltpu.make_async_copy(
                    tbuf.at[u % 2, pl.ds(8 * s, 8), pl.ds(0, _BLK)],
                    out_hbm.at[0, s, wid], osems[u % 2]).wait()
        pltpu.make_async_copy(tok_hbm.at[idx_v.at[l]], gbufs[u],
                              gsems[u]).wait()
        dst = tbuf.at[u % 2]
        gb = gbufs[u]

        @plsc.parallel_loop(0, _BLK, 1, unroll=16)
        def _rows(b):
            bvec = jnp.full((16,), b, dtype=jnp.int32)
            for k in range(4):
                v = gb[b, pl.ds(16 * k, 16)] + pv[k]
                plsc.store_scatter(dst, [dvecs[k], bvec], v)

        for s in range(8):
            pltpu.async_copy(
                tbuf.at[u % 2, pl.ds(8 * s, 8), pl.ds(0, _BLK)],
                out_hbm.at[l, s, wid], osems[u % 2])

    for l in range(_NBUF - 1):
        fire_gather(l, l)

    def step(i, _):
        for u in range(_NBUF):
            substep(_NBUF * i + u, u)
        return _

    lax.fori_loop(0, _L // _NBUF, step, 0, unroll=False)
    for u in range(2):
        for s in range(8):
            pltpu.make_async_copy(
                tbuf.at[u, pl.ds(8 * s, 8), pl.ds(0, _BLK)],
                out_hbm.at[0, s, wid], osems[u]).wait()


@jax.jit
def _tok_pos_embed(xt, token_table, pos_table):
    kfn = functools.partial(
        pl.kernel,
        out_type=jax.ShapeDtypeStruct((_L, 8, _NW, 8, _BLK), jnp.float32),
        mesh=plsc.VectorSubcoreMesh(core_axis_name="c", subcore_axis_name="s"),
        scratch_types=[
            pltpu.VMEM((_L, _BLK), jnp.int32),
            pltpu.VMEM((_L, _D), jnp.float32),
            pltpu.VMEM((_BLK, _D), jnp.float32),
            pltpu.VMEM((_BLK, _D), jnp.float32),
            pltpu.VMEM((_BLK, _D), jnp.float32),
            pltpu.VMEM((_BLK, _D), jnp.float32),
            pltpu.VMEM((2, _D, 129), jnp.float32),
            pltpu.SemaphoreType.DMA,
            pltpu.SemaphoreType.DMA,
            pltpu.SemaphoreType.DMA,
            pltpu.SemaphoreType.DMA,
            pltpu.SemaphoreType.DMA,
            pltpu.SemaphoreType.DMA,
        ],
        compiler_params=pltpu.CompilerParams(use_tc_tiling_on_sc=False,
                                             needs_layout_passes=False),
    )(_sc_body)
    return kfn(xt, token_table, pos_table)


def kernel(x, token_table, pos_table):
    xt = x.astype(jnp.int32).T
    out5 = _tok_pos_embed(xt, token_table, pos_table)
    return out5.transpose(2, 4, 0, 1, 3).reshape(_B, _L, _D)


# speedup vs baseline: 1.0087x; 1.0087x over previous
import functools

import numpy as np

import jax
import jax.numpy as jnp
from jax import lax
from jax.experimental import pallas as pl
from jax.experimental.pallas import tpu as pltpu
from jax.experimental.pallas import tpu_sc as plsc

_NC = 2
_NS = 16
_NW = _NC * _NS

_B = 4096
_L = 200
_D = 64
_BLK = _B // _NW
_NBUF = 4


def _sc_body(xt_hbm, tok2_hbm, pos_hbm, out_hbm,
             idx_v, pos_v, gbuf0, gbuf1, gbuf2, gbuf3, tbuf,
             gsem0, gsem1, gsem2, gsem3, osem0, osem1):
    wid = lax.axis_index("s") * _NC + lax.axis_index("c")
    b0 = wid * _BLK
    tok_hbm = tok2_hbm

    pltpu.sync_copy(xt_hbm.at[:, pl.ds(b0, _BLK)], idx_v)
    pltpu.sync_copy(pos_hbm, pos_v)

    gbufs = (gbuf0, gbuf1, gbuf2, gbuf3)
    gsems = (gsem0, gsem1, gsem2, gsem3)
    osems = (osem0, osem1)

    lanes = lax.iota(jnp.int32, 16)
    dvecs = [16 * k + lanes for k in range(4)]

    def fire_gather(l, slot):
        @pl.when(l < _L)
        def _():
            pltpu.async_copy(tok_hbm.at[idx_v.at[l]], gbufs[slot],
                             gsems[slot])

    def substep(l, u):
        fire_gather(l + _NBUF - 1, (u + _NBUF - 1) % _NBUF)
        pv = [pos_v[l, pl.ds(16 * k, 16)] for k in range(4)]
        @pl.when(l >= 2)
        def _reclaim():
            for s in range(8):
                pltpu.make_async_copy(
                    tbuf.at[u % 2, pl.ds(8 * s, 8), pl.ds(0, _BLK)],
                    out_hbm.at[0, s, wid], osems[u % 2]).wait()
        pltpu.make_async_copy(tok_hbm.at[idx_v.at[l]], gbufs[u],
                              gsems[u]).wait()
        dst = tbuf.at[u % 2]
        gb = gbufs[u]

        @plsc.parallel_loop(0, _BLK, 1, unroll=8)
        def _rows(b):
            bvec = jnp.full((16,), b, dtype=jnp.int32)
            for k in range(4):
                v = gb[b, pl.ds(16 * k, 16)] + pv[k]
                plsc.store_scatter(dst, [dvecs[k], bvec], v)

        for s in range(8):
            pltpu.async_copy(
                tbuf.at[u % 2, pl.ds(8 * s, 8), pl.ds(0, _BLK)],
                out_hbm.at[l, s, wid], osems[u % 2])

    for l in range(_NBUF - 1):
        fire_gather(l, l)

    def step(i, _):
        for u in range(_NBUF):
            substep(_NBUF * i + u, u)
        return _

    lax.fori_loop(0, _L // _NBUF, step, 0, unroll=False)
    for u in range(2):
        for s in range(8):
            pltpu.make_async_copy(
                tbuf.at[u, pl.ds(8 * s, 8), pl.ds(0, _BLK)],
                out_hbm.at[0, s, wid], osems[u]).wait()


@jax.jit
def _tok_pos_embed(xt, token_table, pos_table):
    kfn = functools.partial(
        pl.kernel,
        out_type=jax.ShapeDtypeStruct((_L, 8, _NW, 8, _BLK), jnp.float32),
        mesh=plsc.VectorSubcoreMesh(core_axis_name="c", subcore_axis_name="s"),
        scratch_types=[
            pltpu.VMEM((_L, _BLK), jnp.int32),
            pltpu.VMEM((_L, _D), jnp.float32),
            pltpu.VMEM((_BLK, _D), jnp.float32),
            pltpu.VMEM((_BLK, _D), jnp.float32),
            pltpu.VMEM((_BLK, _D), jnp.float32),
            pltpu.VMEM((_BLK, _D), jnp.float32),
            pltpu.VMEM((2, _D, 129), jnp.float32),
            pltpu.SemaphoreType.DMA,
            pltpu.SemaphoreType.DMA,
            pltpu.SemaphoreType.DMA,
            pltpu.SemaphoreType.DMA,
            pltpu.SemaphoreType.DMA,
            pltpu.SemaphoreType.DMA,
        ],
        compiler_params=pltpu.CompilerParams(use_tc_tiling_on_sc=False,
                                             needs_layout_passes=False),
    )(_sc_body)
    return kfn(xt, token_table, pos_table)


def kernel(x, token_table, pos_table):
    xt = x.astype(jnp.int32).T
    out5 = _tok_pos_embed(xt, token_table, pos_table)
    return out5.transpose(2, 4, 0, 1, 3).reshape(_B, _L, _D)
